# baseline (speedup 1.0000x reference)
import functools

import jax
import jax.numpy as jnp
from jax import lax
from jax.experimental import pallas as pl
from jax.experimental.pallas import tpu as pltpu
from jax.experimental.pallas import tpu_sc as plsc

_B, _S = 4, 8192
_V, _D = 1024, 128
_TOT = _B * _S

_info = plsc.get_sparse_core_info()
_NC, _NS = _info.num_cores, _info.num_subcores
_NW = _NC * _NS
_NPAIR = _NW // 2
_RPP = _TOT // _NPAIR
_CPR = _S // _RPP
_HD = _D // 2
_CH = 128
_NCHUNK = _RPP // _CH
_NG = _RPP // 16


def _digit_masks(ids16):
    dm_b = (ids16 == 1) | ((ids16 >= 17) & (ids16 <= 26))
    om_b = dm_b | (ids16 == 12) | (ids16 == 30)
    one = jnp.full((16,), 1, jnp.int32)
    zero = jnp.full((16,), 0, jnp.int32)
    return jnp.where(dm_b, one, zero), jnp.where(om_b, one, zero)


def _load16(ids_v, g):
    return ids_v[g // 8, pl.ds((g % 8) * 16, 16)]


def _starts(dm, prev_last):
    lane = lax.iota(jnp.int32, 16)
    shifted = dm.at[jnp.maximum(lane - 1, 0)].get(mode="promise_in_bounds")
    prev = jnp.where(lane == 0, jnp.full((16,), prev_last, jnp.int32),
                     shifted)
    return dm * (1 - prev)


def _gather_body(ids_hbm, table_hbm, out_hbm, ids_v, table_v, pos_v,
                 buf0, buf1, isem, tsem, wsem):
    wid = lax.axis_index("s") * _NC + lax.axis_index("c")
    pair = wid // 2
    half = wid % 2
    row = pair // _CPR
    m = pair % _CPR
    base = pair * _RPP
    col0 = half * _HD

    tcopy = pltpu.async_copy(
        table_hbm.at[:, pl.ds(col0, _HD)], table_v, tsem)
    icopy = pltpu.async_copy(ids_hbm.at[row], ids_v, isem)
    icopy.wait()

    def pfx_body(g, carry):
        mvec, prev_last = carry
        ids16 = _load16(ids_v, g)
        dm, _ = _digit_masks(ids16)
        st = _starts(dm, prev_last)
        idxv = jnp.full((16,), g * 16, jnp.int32) + lax.iota(jnp.int32, 16)
        return jnp.maximum(mvec, idxv * st), dm[15]

    mvec0 = jnp.zeros((16,), jnp.int32)
    mvec, prev_last = lax.fori_loop(
        0, m * _NG, pfx_body, (mvec0, jnp.int32(0)))
    runmax = jnp.max(mvec)

    def scan_body(g2, carry):
        runmax, prev_last = carry
        g = m * _NG + g2
        ids16 = _load16(ids_v, g)
        dm, om = _digit_masks(ids16)
        st = _starts(dm, prev_last)
        idxv = jnp.full((16,), g * 16, jnp.int32) + lax.iota(jnp.int32, 16)
        v = idxv * st
        cmax = jnp.maximum(plsc.cummax(v), jnp.full((16,), runmax, jnp.int32))
        pos16 = (idxv - cmax + 1) * om
        pos_v[g2 // 8, pl.ds((g2 % 8) * 16, 16)] = jnp.minimum(pos16, _V - 1)
        return cmax[15], dm[15]

    lax.fori_loop(0, _NG, scan_body, (runmax, prev_last))

    tcopy.wait()

    bufs = (buf0, buf1)
    wcopies = [None] * _NCHUNK
    for c in range(_NCHUNK):
        if c >= 2:
            wcopies[c - 2].wait()
        buf = bufs[c % 2]

        def row_body(g, _, c=c, buf=buf):
            gg = c * (_CH // 16) + g
            posv = pos_v[gg // 8, pl.ds((gg % 8) * 16, 16)]
            for j in range(16):
                r = g * 16 + j
                trow = posv[j]
                for k in range(_HD // 16):
                    buf[r, pl.ds(k * 16, 16)] = \
                        table_v[trow, pl.ds(k * 16, 16)]
            return 0

        lax.fori_loop(0, _CH // 16, row_body, 0, unroll=False)
        wcopies[c] = pltpu.async_copy(
            buf,
            out_hbm.at[pl.ds(base + c * _CH, _CH), pl.ds(col0, _HD)],
            wsem)
    wcopies[_NCHUNK - 2].wait()
    wcopies[_NCHUNK - 1].wait()


_gather = functools.partial(
    pl.kernel,
    mesh=plsc.VectorSubcoreMesh(core_axis_name="c", subcore_axis_name="s"),
    out_type=jax.ShapeDtypeStruct((_TOT, _D), jnp.float32),
    scratch_types=[
        pltpu.VMEM((_S // 128, 128), jnp.int32),
        pltpu.VMEM((_V, _HD), jnp.float32),
        pltpu.VMEM((_NG // 8, 128), jnp.int32),
        pltpu.VMEM((_CH, _HD), jnp.float32),
        pltpu.VMEM((_CH, _HD), jnp.float32),
        pltpu.SemaphoreType.DMA,
        pltpu.SemaphoreType.DMA,
        pltpu.SemaphoreType.DMA,
    ],
    compiler_params=pltpu.CompilerParams(
        use_tc_tiling_on_sc=False, needs_layout_passes=False),
)(_gather_body)


def kernel(input_ids, embedding):
    ids3 = input_ids.reshape(_B, _S // 128, 128)
    out = _gather(ids3, embedding)
    return out.reshape(_B, _S, _D)


# speedup vs baseline: 1.0066x; 1.0066x over previous
import functools

import jax
import jax.numpy as jnp
from jax import lax
from jax.experimental import pallas as pl
from jax.experimental.pallas import tpu as pltpu
from jax.experimental.pallas import tpu_sc as plsc

_B, _S = 4, 8192
_V, _D = 1024, 128
_TOT = _B * _S

_info = plsc.get_sparse_core_info()
_NC, _NS = _info.num_cores, _info.num_subcores
_NW = _NC * _NS
_NPAIR = _NW // 2
_RPP = _TOT // _NPAIR
_CPR = _S // _RPP
_HD = _D // 2
_CH = 128
_NCHUNK = _RPP // _CH
_NG = _RPP // 16


def _digit_masks(ids16):
    dm_b = (ids16 == 1) | ((ids16 >= 17) & (ids16 <= 26))
    om_b = dm_b | (ids16 == 12) | (ids16 == 30)
    one = jnp.full((16,), 1, jnp.int32)
    zero = jnp.full((16,), 0, jnp.int32)
    return jnp.where(dm_b, one, zero), jnp.where(om_b, one, zero)


def _load16(ids_v, g):
    return ids_v[g // 8, pl.ds((g % 8) * 16, 16)]


def _starts(dm, prev_last):
    lane = lax.iota(jnp.int32, 16)
    shifted = dm.at[jnp.maximum(lane - 1, 0)].get(mode="promise_in_bounds")
    prev = jnp.where(lane == 0, jnp.full((16,), prev_last, jnp.int32),
                     shifted)
    return dm * (1 - prev)


def _gather_body(ids_hbm, table_hbm, out_hbm, ids_v, table_v, pos_v,
                 buf0, buf1, isem, tsem, wsem):
    wid = lax.axis_index("s") * _NC + lax.axis_index("c")
    pair = wid // 2
    half = wid % 2
    row = pair // _CPR
    m = pair % _CPR
    base = pair * _RPP
    col0 = half * _HD

    tcopy = pltpu.async_copy(
        table_hbm.at[:, pl.ds(col0, _HD)], table_v, tsem)
    icopy = pltpu.async_copy(ids_hbm.at[row], ids_v, isem)
    icopy.wait()

    def pfx_body(g, carry):
        mvec, prev_last = carry
        ids16 = _load16(ids_v, g)
        dm, _ = _digit_masks(ids16)
        st = _starts(dm, prev_last)
        idxv = jnp.full((16,), g * 16, jnp.int32) + lax.iota(jnp.int32, 16)
        return jnp.maximum(mvec, idxv * st), dm[15]

    mvec0 = jnp.zeros((16,), jnp.int32)
    mvec, prev_last = lax.fori_loop(
        0, m * _NG, pfx_body, (mvec0, jnp.int32(0)))
    runmax = jnp.max(mvec)

    def scan_body(g2, carry):
        runmax, prev_last = carry
        g = m * _NG + g2
        ids16 = _load16(ids_v, g)
        dm, om = _digit_masks(ids16)
        st = _starts(dm, prev_last)
        idxv = jnp.full((16,), g * 16, jnp.int32) + lax.iota(jnp.int32, 16)
        v = idxv * st
        cmax = jnp.maximum(plsc.cummax(v), jnp.full((16,), runmax, jnp.int32))
        pos16 = (idxv - cmax + 1) * om
        pos_v[g2 // 8, pl.ds((g2 % 8) * 16, 16)] = jnp.minimum(pos16, _V - 1)
        return cmax[15], dm[15]

    lax.fori_loop(0, _NG, scan_body, (runmax, prev_last))

    tcopy.wait()

    bufs = (buf0, buf1)
    wcopies = [None] * _NCHUNK
    for c in range(_NCHUNK):
        if c >= 2:
            wcopies[c - 2].wait()
        buf = bufs[c % 2]

        def row_body(g, posv, c=c, buf=buf):
            gg1 = c * (_CH // 16) + g + 1
            nxt = pos_v[gg1 // 8, pl.ds((gg1 % 8) * 16, 16)]
            for j in range(16):
                r = g * 16 + j
                trow = posv[j]
                for k in range(_HD // 16):
                    buf[r, pl.ds(k * 16, 16)] = \
                        table_v[trow, pl.ds(k * 16, 16)]
            return nxt

        gg0 = c * (_CH // 16)
        posv0 = pos_v[gg0 // 8, pl.ds((gg0 % 8) * 16, 16)]
        lax.fori_loop(0, _CH // 16, row_body, posv0, unroll=False)
        wcopies[c] = pltpu.async_copy(
            buf,
            out_hbm.at[pl.ds(base + c * _CH, _CH), pl.ds(col0, _HD)],
            wsem)
    wcopies[_NCHUNK - 2].wait()
    wcopies[_NCHUNK - 1].wait()


_gather = functools.partial(
    pl.kernel,
    mesh=plsc.VectorSubcoreMesh(core_axis_name="c", subcore_axis_name="s"),
    out_type=jax.ShapeDtypeStruct((_TOT, _D), jnp.float32),
    scratch_types=[
        pltpu.VMEM((_S // 128, 128), jnp.int32),
        pltpu.VMEM((_V, _HD), jnp.float32),
        pltpu.VMEM((_NG // 8 + 1, 128), jnp.int32),
        pltpu.VMEM((_CH, _HD), jnp.float32),
        pltpu.VMEM((_CH, _HD), jnp.float32),
        pltpu.SemaphoreType.DMA,
        pltpu.SemaphoreType.DMA,
        pltpu.SemaphoreType.DMA,
    ],
    compiler_params=pltpu.CompilerParams(
        use_tc_tiling_on_sc=False, needs_layout_passes=False),
)(_gather_body)


def kernel(input_ids, embedding):
    ids3 = input_ids.reshape(_B, _S // 128, 128)
    out = _gather(ids3, embedding)
    return out.reshape(_B, _S, _D)


# speedup vs baseline: 2.0265x; 2.0131x over previous
import functools

import jax
import jax.numpy as jnp
from jax import lax
from jax.experimental import pallas as pl
from jax.experimental.pallas import tpu as pltpu
from jax.experimental.pallas import tpu_sc as plsc

_B, _S = 4, 8192
_V, _D = 1024, 128
_TOT = _B * _S

_info = plsc.get_sparse_core_info()
_NC, _NS = _info.num_cores, _info.num_subcores
_NW = _NC * _NS
_BPW = _TOT // _NW
_CH = 128
_NCHUNK = _BPW // _CH


def _pos_body(ids_ref, pos_ref):
    ids = ids_ref[...]
    dm_b = (ids == 1) | ((ids >= 17) & (ids <= 26))
    om = dm_b | (ids == 12) | (ids == 30)
    dm = dm_b.astype(jnp.int32)
    idx = lax.broadcasted_iota(jnp.int32, ids.shape, 1)
    prev = jnp.concatenate(
        [jnp.zeros((ids.shape[0], 1), jnp.int32), dm[:, :-1]], axis=1)
    starts = dm * (1 - prev)
    v = idx * starts
    k = 1
    while k < ids.shape[1]:
        shifted = jnp.concatenate(
            [jnp.zeros((ids.shape[0], k), jnp.int32), v[:, :-k]], axis=1)
        v = jnp.maximum(v, shifted)
        k *= 2
    pos = (idx - v + 1) * om.astype(jnp.int32)
    pos_ref[...] = jnp.clip(pos, 0, _V - 1)


_positions = pl.pallas_call(
    _pos_body,
    out_shape=jax.ShapeDtypeStruct((_B, _S), jnp.int32),
)


def _gather_body(table_hbm, idx_hbm, out_hbm, idx_v, buf0, buf1, stable,
                 isem, wsem, gsem):
    s = lax.axis_index("s")
    wid = s * _NC + lax.axis_index("c")
    base = wid * _BPW

    icopy = pltpu.async_copy(idx_hbm.at[wid], idx_v, isem)

    @pl.when(s == 0)
    def _():
        pltpu.sync_copy(table_hbm, stable)

    plsc.subcore_barrier()
    icopy.wait()

    bufs = (buf0, buf1)
    wcopies = [None] * _NCHUNK
    for c in range(_NCHUNK):
        if c >= 2:
            wcopies[c - 2].wait()
        buf = bufs[c % 2]
        pltpu.async_copy(stable.at[idx_v.at[c]], buf, gsem).wait()
        wcopies[c] = pltpu.async_copy(
            buf, out_hbm.at[pl.ds(base + c * _CH, _CH)], wsem)
    wcopies[_NCHUNK - 2].wait()
    wcopies[_NCHUNK - 1].wait()


_gather = functools.partial(
    pl.kernel,
    mesh=plsc.VectorSubcoreMesh(core_axis_name="c", subcore_axis_name="s"),
    out_type=jax.ShapeDtypeStruct((_TOT, _D), jnp.float32),
    scratch_types=[
        pltpu.VMEM((_NCHUNK, _CH), jnp.int32),
        pltpu.VMEM((_CH, _D), jnp.float32),
        pltpu.VMEM((_CH, _D), jnp.float32),
        pltpu.VMEM_SHARED((_V, _D), jnp.float32),
        pltpu.SemaphoreType.DMA,
        pltpu.SemaphoreType.DMA,
        pltpu.SemaphoreType.DMA,
    ],
    compiler_params=pltpu.CompilerParams(use_tc_tiling_on_sc=False),
)(_gather_body)


def kernel(input_ids, embedding):
    pos = _positions(input_ids)
    pos = pos.reshape(_NW, _NCHUNK, _CH)
    out = _gather(embedding, pos)
    return out.reshape(_B, _S, _D)


# speedup vs baseline: 2.4157x; 1.1921x over previous
import functools

import jax
import jax.numpy as jnp
from jax import lax
from jax.experimental import pallas as pl
from jax.experimental.pallas import tpu as pltpu
from jax.experimental.pallas import tpu_sc as plsc

_B, _S = 4, 8192
_V, _D = 1024, 128
_TOT = _B * _S

_info = plsc.get_sparse_core_info()
_NC, _NS = _info.num_cores, _info.num_subcores
_NW = _NC * _NS
_BPW = _TOT // _NW
_CH = 128
_NCHUNK = _BPW // _CH


def _pos_body(ids_ref, pos_ref):
    ids = ids_ref[...]
    dm_b = (ids == 1) | ((ids >= 17) & (ids <= 26))
    om = dm_b | (ids == 12) | (ids == 30)
    dm = dm_b.astype(jnp.int32)
    idx = lax.broadcasted_iota(jnp.int32, ids.shape, 1)
    prev = jnp.concatenate(
        [jnp.zeros((ids.shape[0], 1), jnp.int32), dm[:, :-1]], axis=1)
    starts = dm * (1 - prev)
    v = idx * starts
    k = 1
    while k < ids.shape[1]:
        shifted = jnp.concatenate(
            [jnp.zeros((ids.shape[0], k), jnp.int32), v[:, :-k]], axis=1)
        v = jnp.maximum(v, shifted)
        k *= 2
    pos = (idx - v + 1) * om.astype(jnp.int32)
    pos_ref[...] = jnp.clip(pos, 0, _V - 1)


_positions = pl.pallas_call(
    _pos_body,
    out_shape=jax.ShapeDtypeStruct((_B, _S), jnp.int32),
)


def _gather_body(table_hbm, idx_hbm, out_hbm, idx_v, buf0, buf1, buf2,
                 stable, isem, wsem, gsem):
    s = lax.axis_index("s")
    wid = s * _NC + lax.axis_index("c")
    base = wid * _BPW

    icopy = pltpu.async_copy(idx_hbm.at[wid], idx_v, isem)

    @pl.when(s == 0)
    def _():
        pltpu.sync_copy(table_hbm, stable)

    plsc.subcore_barrier()
    icopy.wait()

    bufs = (buf0, buf1, buf2)
    wcopies = [None] * _NCHUNK
    gcopies = [None] * _NCHUNK
    for c in range(2):
        gcopies[c] = pltpu.async_copy(
            stable.at[idx_v.at[c]], bufs[c % 3], gsem)
    for c in range(_NCHUNK):
        gcopies[c].wait()
        wcopies[c] = pltpu.async_copy(
            bufs[c % 3], out_hbm.at[pl.ds(base + c * _CH, _CH)], wsem)
        if c + 2 < _NCHUNK:
            if c >= 1:
                wcopies[c - 1].wait()
            gcopies[c + 2] = pltpu.async_copy(
                stable.at[idx_v.at[c + 2]], bufs[(c + 2) % 3], gsem)
    wcopies[_NCHUNK - 2].wait()
    wcopies[_NCHUNK - 1].wait()


_gather = functools.partial(
    pl.kernel,
    mesh=plsc.VectorSubcoreMesh(core_axis_name="c", subcore_axis_name="s"),
    out_type=jax.ShapeDtypeStruct((_TOT, _D), jnp.float32),
    scratch_types=[
        pltpu.VMEM((_NCHUNK, _CH), jnp.int32),
        pltpu.VMEM((_CH, _D), jnp.float32),
        pltpu.VMEM((_CH, _D), jnp.float32),
        pltpu.VMEM((_CH, _D), jnp.float32),
        pltpu.VMEM_SHARED((_V, _D), jnp.float32),
        pltpu.SemaphoreType.DMA,
        pltpu.SemaphoreType.DMA,
        pltpu.SemaphoreType.DMA,
    ],
    compiler_params=pltpu.CompilerParams(use_tc_tiling_on_sc=False),
)(_gather_body)


def kernel(input_ids, embedding):
    pos = _positions(input_ids)
    pos = pos.reshape(_NW, _NCHUNK, _CH)
    out = _gather(embedding, pos)
    return out.reshape(_B, _S, _D)
